# baseline (speedup 1.0000x reference)
import functools

import jax
import jax.numpy as jnp
from jax import lax
from jax.experimental import pallas as pl
from jax.experimental.pallas import tpu as pltpu
from jax.experimental.pallas import tpu_sc as plsc

_NBUF = 4


def _emb_kernel(bsz, seq, d, n_workers, num_cores, num_subcores, chunk):
    n_per_w = (bsz * seq) // n_workers
    w_per_b = n_workers // bsz
    n_chunks = n_per_w // chunk
    assert n_chunks % _NBUF == 0 and n_chunks >= 3 * _NBUF

    mesh = plsc.VectorSubcoreMesh(core_axis_name="c", subcore_axis_name="s")

    @functools.partial(
        pl.kernel,
        mesh=mesh,
        out_type=jax.ShapeDtypeStruct((bsz, seq, d), jnp.float32),
        scratch_types=[
            pltpu.VMEM((n_per_w,), jnp.int32),
            pltpu.VMEM((_NBUF, chunk, d), jnp.float32),
            pltpu.VMEM_SHARED((num_subcores, 2, chunk, d), jnp.float32),
        ]
        + [pltpu.SemaphoreType.DMA] * (_NBUF + 6),
    )
    def emb(idx_hbm, table_hbm, out_hbm, idx_v, rows_v, sp_v, *sems):
        sg = sems[:_NBUF]
        sd = sems[_NBUF : _NBUF + 2]
        sx = sems[_NBUF + 2 : _NBUF + 4]
        sp2 = sems[_NBUF + 4 :]
        cid = lax.axis_index("c")
        sid = lax.axis_index("s")
        wid = sid * num_cores + cid
        batch = wid // w_per_b
        off = (wid % w_per_b) * n_per_w
        pltpu.sync_copy(idx_hbm.at[batch, pl.ds(off, n_per_w)], idx_v)

        def out_slice(i):
            return out_hbm.at[batch, pl.ds(off + i * chunk, chunk)]

        def gather(i, b):
            return pltpu.make_async_copy(
                table_hbm.at[idx_v.at[pl.ds(i * chunk, chunk)]],
                rows_v.at[b],
                sg[b],
            )

        def put_direct(i, b):
            return pltpu.make_async_copy(rows_v.at[b], out_slice(i), sd[b // 2])

        def xbar(b):
            m = (b - 1) // 2
            return pltpu.make_async_copy(rows_v.at[b], sp_v.at[sid, m], sx[m])

        def put_sp(i, b):
            m = (b - 1) // 2
            return pltpu.make_async_copy(sp_v.at[sid, m], out_slice(i), sp2[m])

        gather(0, 0).start()
        gather(1, 1).start()

        def body(k, carry):
            for b in range(_NBUF):
                i = _NBUF * k + b
                nb = (b + 2) % _NBUF

                @pl.when(i >= 2)
                def _():
                    if nb % 2 == 0:
                        put_direct(i - 2, nb).wait()
                    else:
                        xbar(nb).wait()
                        put_sp(i - 2, nb).start()

                @pl.when(i + 2 < n_chunks)
                def _():
                    gather(i + 2, nb).start()

                gather(i, b).wait()
                if b % 2 == 0:
                    put_direct(i, b).start()
                else:
                    @pl.when(i >= _NBUF)
                    def _():
                        put_sp(i - _NBUF, b).wait()

                    xbar(b).start()
            return carry

        lax.fori_loop(0, n_chunks // _NBUF, body, 0)
        put_direct(n_chunks - 2, 2).wait()
        xbar(3).wait()
        put_sp(n_chunks - 1, 3).start()
        put_sp(n_chunks - 3, 1).wait()
        put_sp(n_chunks - 1, 3).wait()

    return emb


def kernel(input_ids, table):
    b, s = input_ids.shape
    v, d = table.shape
    idx = input_ids.astype(jnp.int32)
    info = plsc.get_sparse_core_info()
    nw = info.num_cores * info.num_subcores
    emb = _emb_kernel(
        b, s, d, nw, info.num_cores, info.num_subcores, chunk=8
    )
    return emb(idx, table)
